# Initial kernel scaffold; baseline (speedup 1.0000x reference)
import jax
import jax.numpy as jnp
from jax.experimental import pallas as pl


def kernel(h, e, edge_index, WA, bA, WB, bB, WC, bC, WD, bD, WE, bE, gamma_h, beta_h, gamma_e, beta_e):
    raise NotImplementedError("write your pallas kernel here")


# speedup vs baseline: 2.4543x; 2.4543x over previous
import jax
import jax.numpy as jnp
from jax import lax
from jax.experimental import pallas as pl
from jax.experimental.pallas import tpu as pltpu
from jax.experimental.pallas import tpu_sc as plsc

_N = 10000
_E = 320000
_D = 128
_H = _D // 2
_NS = 16
_NC = 2
_EPW = _E // _NS
_B = 80
_NB = _EPW // _B
_NZ = 624
_NR = _N - _NS * _NZ
_L = 16


def _node_mm_body(h_ref, wa_ref, ba_ref, wb_ref, bb_ref, wd_ref, bd_ref,
                  we_ref, be_ref, ah_ref, db_ref, eh_ref):
    h = h_ref[...]
    ah_ref[...] = jnp.dot(h, wa_ref[...], preferred_element_type=jnp.float32) + ba_ref[...]
    bh = jnp.dot(h, wb_ref[...], preferred_element_type=jnp.float32) + bb_ref[...]
    dh = jnp.dot(h, wd_ref[...], preferred_element_type=jnp.float32) + bd_ref[...]
    eh = jnp.dot(h, we_ref[...], preferred_element_type=jnp.float32) + be_ref[...]
    db_ref[0] = jnp.concatenate([dh[:, :_H], bh[:, :_H]], axis=1)
    db_ref[1] = jnp.concatenate([dh[:, _H:], bh[:, _H:]], axis=1)
    eh_ref[0] = eh
    eh_ref[1] = jnp.concatenate([eh[:, _H:], eh[:, :_H]], axis=1)


def _node_mm(h, wa, ba, wb, bb, wd, bd, we, be):
    out_shape = [
        jax.ShapeDtypeStruct((_N, _D), jnp.float32),
        jax.ShapeDtypeStruct((_NC, _N, _D), jnp.float32),
        jax.ShapeDtypeStruct((_NC, _N, _D), jnp.float32),
    ]
    return pl.pallas_call(_node_mm_body, out_shape=out_shape)(
        h, wa, ba, wb, bb, wd, bd, we, be)


_BE = 2000


def _ce_body(e_ref, wc_ref, bc_ref, ce_ref):
    ce = jnp.dot(e_ref[...], wc_ref[...], preferred_element_type=jnp.float32) + bc_ref[...]
    ce_ref[0] = ce[:, :_H]
    ce_ref[1] = ce[:, _H:]


def _ce_mm(e, wc, bc):
    return pl.pallas_call(
        _ce_body,
        grid=(_E // _BE,),
        in_specs=[
            pl.BlockSpec((_BE, _D), lambda i: (i, 0)),
            pl.BlockSpec((_D, _D), lambda i: (0, 0)),
            pl.BlockSpec((_D,), lambda i: (0,)),
        ],
        out_specs=pl.BlockSpec((_NC, _BE, _H), lambda i: (0, i, 0)),
        out_shape=jax.ShapeDtypeStruct((_NC, _E, _H), jnp.float32),
    )(e, wc, bc)


def _sc_edge_body(src, dst, db2, eh2, ce2, zeros,
                  enew, acc_o, stats,
                  srco_v, dstr_v, db_v, eh_v, en_v, sums_v,
                  acc, semd, seme, semc):
    c = lax.axis_index("c")
    s = lax.axis_index("s")

    z0 = s * _NZ
    pltpu.sync_copy(zeros.at[pl.ds(0, _NZ)], acc.at[pl.ds(z0, _NZ)])

    @pl.when(s == _NS - 1)
    def _zero_tail():
        pltpu.sync_copy(zeros.at[pl.ds(0, _NR)], acc.at[pl.ds(_NS * _NZ, _NR)])

    plsc.subcore_barrier()

    base = s * _EPW
    zv = jnp.zeros((_L,), jnp.float32)
    init = (zv,) * 8

    def outer(b, carry):
        off = base + b * _B
        pltpu.sync_copy(src.at[pl.ds(off, _B)], srco_v.at[0])
        pltpu.sync_copy(dst.at[pl.ds(off, _B)], dstr_v.at[0])
        w0 = pltpu.async_copy(db2.at[c].at[srco_v.at[0]], db_v, semd)
        w1 = pltpu.async_copy(eh2.at[c].at[dstr_v.at[0]], eh_v, seme)
        w2 = pltpu.async_copy(ce2.at[c, pl.ds(off, _B)], en_v, semc)
        w0.wait()
        w1.wait()
        w2.wait()

        def row(r, t):
            t = list(t)
            for j in range(_H // _L):
                sl = pl.ds(j * _L, _L)
                sh = pl.ds(_H + j * _L, _L)
                en = db_v[r, sl] + eh_v[r, sl] + en_v[r, sl]
                en_v[r, sl] = en
                sg = 1.0 / (1.0 + jnp.exp(-en))
                msg = db_v[r, sh] * sg
                db_v[r, sl] = msg
                db_v[r, sh] = sg
                t[j] = t[j] + en
                t[4 + j] = t[4 + j] + en * en
            return tuple(t)

        carry = lax.fori_loop(0, _B, row, carry)

        pltpu.sync_copy(en_v, enew.at[c, pl.ds(off, _B)])
        pltpu.sync_copy(db_v, acc.at[dstr_v.at[0]], add=True)
        return carry

    carry = lax.fori_loop(0, _NB, outer, init)

    for j in range(_H // _L):
        sums_v[0, pl.ds(j * _L, _L)] = carry[j]
        sums_v[1, pl.ds(j * _L, _L)] = carry[4 + j]
    pltpu.sync_copy(sums_v, stats.at[c, s])

    plsc.subcore_barrier()
    pltpu.sync_copy(acc.at[pl.ds(z0, _NZ)], acc_o.at[c, pl.ds(z0, _NZ)])

    @pl.when(s == _NS - 1)
    def _out_tail():
        t0 = _NS * _NZ
        pltpu.sync_copy(acc.at[pl.ds(t0, _NR)], acc_o.at[c, pl.ds(t0, _NR)])


def _sc_edge(src, dst, db2, eh2, ce2, zeros):
    mesh = plsc.VectorSubcoreMesh(core_axis_name="c", subcore_axis_name="s")
    f = pl.kernel(
        _sc_edge_body,
        out_type=[
            jax.ShapeDtypeStruct((_NC, _E, _H), jnp.float32),
            jax.ShapeDtypeStruct((_NC, _N, _D), jnp.float32),
            jax.ShapeDtypeStruct((_NC, _NS, 2, _H), jnp.float32),
        ],
        mesh=mesh,
        scratch_types=[
            pltpu.VMEM((1, _B), jnp.int32),
            pltpu.VMEM((1, _B), jnp.int32),
            pltpu.VMEM((_B, _D), jnp.float32),
            pltpu.VMEM((_B, _D), jnp.float32),
            pltpu.VMEM((_B, _H), jnp.float32),
            pltpu.VMEM((2, _H), jnp.float32),
            pltpu.VMEM_SHARED((_N, _D), jnp.float32),
            pltpu.SemaphoreType.DMA,
            pltpu.SemaphoreType.DMA,
            pltpu.SemaphoreType.DMA,
        ],
    )
    return f(src, dst, db2, eh2, ce2, zeros)


def _hfin_body(ah_ref, acc_ref, h_ref, gh_ref, bh_ref, st_ref,
               ge_ref, be_ref, ho_ref, scale_ref, shift_ref):
    acch = jnp.concatenate([acc_ref[0, :, :_H], acc_ref[1, :, :_H]], axis=1)
    accs = jnp.concatenate([acc_ref[0, :, _H:], acc_ref[1, :, _H:]], axis=1)
    q = ah_ref[...] + acch / (accs + 1e-6)
    m = jnp.mean(q, axis=0, keepdims=True)
    v = jnp.mean((q - m) * (q - m), axis=0, keepdims=True)
    hn = gh_ref[...] * (q - m) / jnp.sqrt(v + 1e-5) + bh_ref[...]
    ho_ref[...] = h_ref[...] + jnp.maximum(hn, 0.0)

    stc = jnp.sum(st_ref[...], axis=1)
    mean = jnp.concatenate([stc[0, 0], stc[1, 0]]) * (1.0 / _E)
    msq = jnp.concatenate([stc[0, 1], stc[1, 1]]) * (1.0 / _E)
    var = msq - mean * mean
    scale = ge_ref[...] / jnp.sqrt(var + 1e-5)
    scale_ref[...] = scale.reshape(1, _D)
    shift_ref[...] = (be_ref[...] - mean * scale).reshape(1, _D)


def _hfin(ah, acc, h, gamma_h, beta_h, stats, gamma_e, beta_e):
    out_shape = [
        jax.ShapeDtypeStruct((_N, _D), jnp.float32),
        jax.ShapeDtypeStruct((1, _D), jnp.float32),
        jax.ShapeDtypeStruct((1, _D), jnp.float32),
    ]
    return pl.pallas_call(_hfin_body, out_shape=out_shape)(
        ah, acc, h, gamma_h, beta_h, stats, gamma_e, beta_e)


_BF = 2000


def _efin_body(e_ref, en_ref, scale_ref, shift_ref, eo_ref):
    en = jnp.concatenate([en_ref[0], en_ref[1]], axis=1)
    eo_ref[...] = e_ref[...] + jnp.maximum(
        en * scale_ref[...] + shift_ref[...], 0.0)


def _efin(e, enew, scale, shift):
    return pl.pallas_call(
        _efin_body,
        grid=(_E // _BF,),
        in_specs=[
            pl.BlockSpec((_BF, _D), lambda i: (i, 0)),
            pl.BlockSpec((_NC, _BF, _H), lambda i: (0, i, 0)),
            pl.BlockSpec((1, _D), lambda i: (0, 0)),
            pl.BlockSpec((1, _D), lambda i: (0, 0)),
        ],
        out_specs=pl.BlockSpec((_BF, _D), lambda i: (i, 0)),
        out_shape=jax.ShapeDtypeStruct((_E, _D), jnp.float32),
    )(e, enew, scale, shift)


def kernel(h, e, edge_index, WA, bA, WB, bB, WC, bC, WD, bD, WE, bE,
           gamma_h, beta_h, gamma_e, beta_e):
    src = edge_index[0]
    dst = edge_index[1]
    zeros = jnp.zeros((_NZ, _D), jnp.float32)

    ah, db2, eh2 = _node_mm(h, WA, bA, WB, bB, WD, bD, WE, bE)
    ce2 = _ce_mm(e, WC, bC)

    enew, acc, stats = _sc_edge(src, dst, db2, eh2, ce2, zeros)

    h_out, scale, shift = _hfin(ah, acc, h, gamma_h, beta_h,
                                stats, gamma_e, beta_e)
    e_out = _efin(e, enew, scale, shift)
    return (h_out, e_out)


# speedup vs baseline: 3.2155x; 1.3102x over previous
import jax
import jax.numpy as jnp
from jax import lax
from jax.experimental import pallas as pl
from jax.experimental.pallas import tpu as pltpu
from jax.experimental.pallas import tpu_sc as plsc

_N = 10000
_E = 320000
_D = 128
_H = _D // 2
_NS = 16
_NC = 2
_EPW = _E // _NS
_B = 40
_NB = _EPW // _B
_NZ = 624
_NR = _N - _NS * _NZ
_ZB = 16
_L = 16


def _node_mm_body(h_ref, wa_ref, ba_ref, wb_ref, bb_ref, wd_ref, bd_ref,
                  we_ref, be_ref, ah_ref, db_ref, eh_ref):
    h = h_ref[...]
    ah_ref[...] = jnp.dot(h, wa_ref[...], preferred_element_type=jnp.float32) + ba_ref[...]
    bh = jnp.dot(h, wb_ref[...], preferred_element_type=jnp.float32) + bb_ref[...]
    dh = jnp.dot(h, wd_ref[...], preferred_element_type=jnp.float32) + bd_ref[...]
    eh = jnp.dot(h, we_ref[...], preferred_element_type=jnp.float32) + be_ref[...]
    db_ref[0] = jnp.concatenate([dh[:, :_H], bh[:, :_H]], axis=1)
    db_ref[1] = jnp.concatenate([dh[:, _H:], bh[:, _H:]], axis=1)
    eh_ref[0] = eh
    eh_ref[1] = jnp.concatenate([eh[:, _H:], eh[:, :_H]], axis=1)


def _node_mm(h, wa, ba, wb, bb, wd, bd, we, be):
    out_shape = [
        jax.ShapeDtypeStruct((_N, _D), jnp.float32),
        jax.ShapeDtypeStruct((_NC, _N, _D), jnp.float32),
        jax.ShapeDtypeStruct((_NC, _N, _D), jnp.float32),
    ]
    return pl.pallas_call(_node_mm_body, out_shape=out_shape)(
        h, wa, ba, wb, bb, wd, bd, we, be)


_BE = 2000


def _ce_body(e_ref, wc_ref, bc_ref, ce_ref):
    ce = jnp.dot(e_ref[...], wc_ref[...], preferred_element_type=jnp.float32) + bc_ref[...]
    ce_ref[0] = ce[:, :_H]
    ce_ref[1] = ce[:, _H:]


def _ce_mm(e, wc, bc):
    return pl.pallas_call(
        _ce_body,
        grid=(_E // _BE,),
        in_specs=[
            pl.BlockSpec((_BE, _D), lambda i: (i, 0)),
            pl.BlockSpec((_D, _D), lambda i: (0, 0)),
            pl.BlockSpec((_D,), lambda i: (0,)),
        ],
        out_specs=pl.BlockSpec((_NC, _BE, _H), lambda i: (0, i, 0)),
        out_shape=jax.ShapeDtypeStruct((_NC, _E, _H), jnp.float32),
    )(e, wc, bc)


def _sc_edge_body(src, dst, db2, eh2, ce2, zeros,
                  enew, acc_o, stats,
                  srco0, dstr0, db0, eh0, en0,
                  srco1, dstr1, db1, eh1, en1, sums_v,
                  acc, g0, g1, w0, w1):
    c = lax.axis_index("c")
    s = lax.axis_index("s")
    sets = ((srco0, dstr0, db0, eh0, en0, g0, w0),
            (srco1, dstr1, db1, eh1, en1, g1, w1))

    z0 = s * _NZ

    def zrow(i, _):
        pltpu.sync_copy(zeros, acc.at[pl.ds(z0 + i * _ZB, _ZB)])
        return 0

    lax.fori_loop(0, _NZ // _ZB, zrow, 0)

    @pl.when(s == _NS - 1)
    def _zero_tail():
        pltpu.sync_copy(zeros, acc.at[pl.ds(_NS * _NZ, _NR)])

    plsc.subcore_barrier()

    base = s * _EPW
    zv = jnp.zeros((_L,), jnp.float32)
    init = (zv,) * 8

    def stage(bidx, st):
        srco, dstr, db_v, eh_v, en_v, g, _ = st
        off = base + bidx * _B
        pltpu.sync_copy(src.at[pl.ds(off, _B)], srco.at[0])
        pltpu.sync_copy(dst.at[pl.ds(off, _B)], dstr.at[0])
        pltpu.async_copy(db2.at[c].at[srco.at[0]], db_v, g)
        pltpu.async_copy(eh2.at[c].at[dstr.at[0]], eh_v, g)
        pltpu.async_copy(ce2.at[c, pl.ds(off, _B)], en_v, g)

    def wait_gathers(st):
        srco, dstr, db_v, eh_v, en_v, g, _ = st
        pltpu.make_async_copy(db2.at[c].at[srco.at[0]], db_v, g).wait()
        pltpu.make_async_copy(eh2.at[c].at[dstr.at[0]], eh_v, g).wait()
        pltpu.make_async_copy(ce2.at[c, pl.ds(0, _B)], en_v, g).wait()

    def put(bidx, st):
        _, dstr, db_v, _, en_v, _, w = st
        off = base + bidx * _B
        pltpu.async_copy(en_v, enew.at[c, pl.ds(off, _B)], w)
        pltpu.sync_copy(db_v, acc.at[dstr.at[0]], add=True)

    def wait_put(st):
        _, _, _, _, en_v, _, w = st
        pltpu.make_async_copy(en_v, enew.at[c, pl.ds(0, _B)], w).wait()

    def compute(st, carry):
        _, _, db_v, eh_v, en_v, _, _ = st

        def row(r2, t):
            t = list(t)
            for u in range(2):
                r = r2 * 2 + u
                for j in range(_H // _L):
                    sl = pl.ds(j * _L, _L)
                    sh = pl.ds(_H + j * _L, _L)
                    en = db_v[r, sl] + eh_v[r, sl] + en_v[r, sl]
                    en_v[r, sl] = en
                    sg = 1.0 / (1.0 + jnp.exp(-en))
                    msg = db_v[r, sh] * sg
                    db_v[r, sl] = msg
                    db_v[r, sh] = sg
                    t[j] = t[j] + en
                    t[4 + j] = t[4 + j] + en * en
            return tuple(t)

        return lax.fori_loop(0, _B // 2, row, carry)

    stage(0, sets[0])

    def outer(i, carry):
        a = 2 * i

        @pl.when(i > 0)
        def _drain1():
            wait_put(sets[1])

        stage(a + 1, sets[1])
        wait_gathers(sets[0])
        carry = compute(sets[0], carry)
        put(a, sets[0])
        wait_put(sets[0])

        @pl.when(a + 2 < _NB)
        def _stage0():
            stage(a + 2, sets[0])

        wait_gathers(sets[1])
        carry = compute(sets[1], carry)
        put(a + 1, sets[1])
        return carry

    carry = lax.fori_loop(0, _NB // 2, outer, init)
    wait_put(sets[1])

    for j in range(_H // _L):
        sums_v[0, pl.ds(j * _L, _L)] = carry[j]
        sums_v[1, pl.ds(j * _L, _L)] = carry[4 + j]
    pltpu.sync_copy(sums_v, stats.at[c, s])

    plsc.subcore_barrier()
    pltpu.sync_copy(acc.at[pl.ds(z0, _NZ)], acc_o.at[c, pl.ds(z0, _NZ)])

    @pl.when(s == _NS - 1)
    def _out_tail():
        t0 = _NS * _NZ
        pltpu.sync_copy(acc.at[pl.ds(t0, _NR)], acc_o.at[c, pl.ds(t0, _NR)])


def _sc_edge(src, dst, db2, eh2, ce2, zeros):
    mesh = plsc.VectorSubcoreMesh(core_axis_name="c", subcore_axis_name="s")
    f = pl.kernel(
        _sc_edge_body,
        out_type=[
            jax.ShapeDtypeStruct((_NC, _E, _H), jnp.float32),
            jax.ShapeDtypeStruct((_NC, _N, _D), jnp.float32),
            jax.ShapeDtypeStruct((_NC, _NS, 2, _H), jnp.float32),
        ],
        mesh=mesh,
        scratch_types=(
            [pltpu.VMEM((1, _B), jnp.int32),
             pltpu.VMEM((1, _B), jnp.int32),
             pltpu.VMEM((_B, _D), jnp.float32),
             pltpu.VMEM((_B, _D), jnp.float32),
             pltpu.VMEM((_B, _H), jnp.float32)] * 2
            + [pltpu.VMEM((2, _H), jnp.float32),
               pltpu.VMEM_SHARED((_N, _D), jnp.float32),
               pltpu.SemaphoreType.DMA,
               pltpu.SemaphoreType.DMA,
               pltpu.SemaphoreType.DMA,
               pltpu.SemaphoreType.DMA]
        ),
    )
    return f(src, dst, db2, eh2, ce2, zeros)


def _hfin_body(ah_ref, acc_ref, h_ref, gh_ref, bh_ref, st_ref,
               ge_ref, be_ref, ho_ref, scale_ref, shift_ref):
    acch = jnp.concatenate([acc_ref[0, :, :_H], acc_ref[1, :, :_H]], axis=1)
    accs = jnp.concatenate([acc_ref[0, :, _H:], acc_ref[1, :, _H:]], axis=1)
    q = ah_ref[...] + acch / (accs + 1e-6)
    m = jnp.mean(q, axis=0, keepdims=True)
    v = jnp.mean((q - m) * (q - m), axis=0, keepdims=True)
    hn = gh_ref[...] * (q - m) / jnp.sqrt(v + 1e-5) + bh_ref[...]
    ho_ref[...] = h_ref[...] + jnp.maximum(hn, 0.0)

    stc = jnp.sum(st_ref[...], axis=1)
    mean = jnp.concatenate([stc[0, 0], stc[1, 0]]) * (1.0 / _E)
    msq = jnp.concatenate([stc[0, 1], stc[1, 1]]) * (1.0 / _E)
    var = msq - mean * mean
    scale = ge_ref[...] / jnp.sqrt(var + 1e-5)
    scale_ref[...] = scale.reshape(1, _D)
    shift_ref[...] = (be_ref[...] - mean * scale).reshape(1, _D)


def _hfin(ah, acc, h, gamma_h, beta_h, stats, gamma_e, beta_e):
    out_shape = [
        jax.ShapeDtypeStruct((_N, _D), jnp.float32),
        jax.ShapeDtypeStruct((1, _D), jnp.float32),
        jax.ShapeDtypeStruct((1, _D), jnp.float32),
    ]
    return pl.pallas_call(_hfin_body, out_shape=out_shape)(
        ah, acc, h, gamma_h, beta_h, stats, gamma_e, beta_e)


_BF = 2000


def _efin_body(e_ref, en_ref, scale_ref, shift_ref, eo_ref):
    en = jnp.concatenate([en_ref[0], en_ref[1]], axis=1)
    eo_ref[...] = e_ref[...] + jnp.maximum(
        en * scale_ref[...] + shift_ref[...], 0.0)


def _efin(e, enew, scale, shift):
    return pl.pallas_call(
        _efin_body,
        grid=(_E // _BF,),
        in_specs=[
            pl.BlockSpec((_BF, _D), lambda i: (i, 0)),
            pl.BlockSpec((_NC, _BF, _H), lambda i: (0, i, 0)),
            pl.BlockSpec((1, _D), lambda i: (0, 0)),
            pl.BlockSpec((1, _D), lambda i: (0, 0)),
        ],
        out_specs=pl.BlockSpec((_BF, _D), lambda i: (i, 0)),
        out_shape=jax.ShapeDtypeStruct((_E, _D), jnp.float32),
    )(e, enew, scale, shift)


def kernel(h, e, edge_index, WA, bA, WB, bB, WC, bC, WD, bD, WE, bE,
           gamma_h, beta_h, gamma_e, beta_e):
    src = edge_index[0]
    dst = edge_index[1]
    zeros = jnp.zeros((_ZB, _D), jnp.float32)

    ah, db2, eh2 = _node_mm(h, WA, bA, WB, bB, WD, bD, WE, bE)
    ce2 = _ce_mm(e, WC, bC)

    enew, acc, stats = _sc_edge(src, dst, db2, eh2, ce2, zeros)

    h_out, scale, shift = _hfin(ah, acc, h, gamma_h, beta_h,
                                stats, gamma_e, beta_e)
    e_out = _efin(e, enew, scale, shift)
    return (h_out, e_out)


# speedup vs baseline: 3.9976x; 1.2432x over previous
import jax
import jax.numpy as jnp
from jax import lax
from jax.experimental import pallas as pl
from jax.experimental.pallas import tpu as pltpu
from jax.experimental.pallas import tpu_sc as plsc

_N = 10000
_E = 320000
_D = 128
_H = _D // 2
_NS = 16
_NC = 2
_EPW = _E // _NS
_B = 40
_NB = _EPW // _B
_NZ = 624
_NR = _N - _NS * _NZ
_ZB = 16
_L = 16


def _node_mm_body(h_ref, wa_ref, ba_ref, wb_ref, bb_ref, wd_ref, bd_ref,
                  we_ref, be_ref, ah_ref, db_ref, eh_ref):
    h = h_ref[...]
    ah_ref[...] = jnp.dot(h, wa_ref[...], preferred_element_type=jnp.float32) + ba_ref[...]
    bh = jnp.dot(h, wb_ref[...], preferred_element_type=jnp.float32) + bb_ref[...]
    dh = jnp.dot(h, wd_ref[...], preferred_element_type=jnp.float32) + bd_ref[...]
    eh = jnp.dot(h, we_ref[...], preferred_element_type=jnp.float32) + be_ref[...]
    db_ref[0] = jnp.concatenate([dh[:, :_H], bh[:, :_H]], axis=1)
    db_ref[1] = jnp.concatenate([dh[:, _H:], bh[:, _H:]], axis=1)
    eh_ref[0] = eh
    eh_ref[1] = jnp.concatenate([eh[:, _H:], eh[:, :_H]], axis=1)


def _interleave_bf16(x):
    n = x.shape[-2]
    y = x.reshape(_NC, n, _D // 32, 2, 16)
    y = jnp.swapaxes(y, -1, -2).astype(jnp.bfloat16)
    return jax.lax.bitcast_convert_type(y, jnp.int32).reshape(_NC, n, _D // 2)


def _node_mm(h, wa, ba, wb, bb, wd, bd, we, be):
    out_shape = [
        jax.ShapeDtypeStruct((_N, _D), jnp.float32),
        jax.ShapeDtypeStruct((_NC, _N, _D), jnp.float32),
        jax.ShapeDtypeStruct((_NC, _N, _D), jnp.float32),
    ]
    return pl.pallas_call(_node_mm_body, out_shape=out_shape)(
        h, wa, ba, wb, bb, wd, bd, we, be)


_BE = 2000


def _ce_body(e_ref, wc_ref, bc_ref, ce_ref):
    ce = jnp.dot(e_ref[...], wc_ref[...], preferred_element_type=jnp.float32) + bc_ref[...]
    ce_ref[0] = ce[:, :_H]
    ce_ref[1] = ce[:, _H:]


def _ce_mm(e, wc, bc):
    return pl.pallas_call(
        _ce_body,
        grid=(_E // _BE,),
        in_specs=[
            pl.BlockSpec((_BE, _D), lambda i: (i, 0)),
            pl.BlockSpec((_D, _D), lambda i: (0, 0)),
            pl.BlockSpec((_D,), lambda i: (0,)),
        ],
        out_specs=pl.BlockSpec((_NC, _BE, _H), lambda i: (0, i, 0)),
        out_shape=jax.ShapeDtypeStruct((_NC, _E, _H), jnp.float32),
    )(e, wc, bc)


def _sc_edge_body(src, dst, db2, eh2, ce2, zeros,
                  enew, acc_o, stats,
                  srco0, dstr0, db0, eh0, en0,
                  srco1, dstr1, db1, eh1, en1, sums_v,
                  acc, g0, g1, w0, w1, i0, i1):
    c = lax.axis_index("c")
    s = lax.axis_index("s")
    sets = ((srco0, dstr0, db0, eh0, en0, g0, w0, i0),
            (srco1, dstr1, db1, eh1, en1, g1, w1, i1))

    z0 = s * _NZ

    def zrow(i, _):
        pltpu.sync_copy(zeros, acc.at[pl.ds(z0 + i * _ZB, _ZB)])
        return 0

    lax.fori_loop(0, _NZ // _ZB, zrow, 0)

    @pl.when(s == _NS - 1)
    def _zero_tail():
        pltpu.sync_copy(zeros, acc.at[pl.ds(_NS * _NZ, _NR)])

    plsc.subcore_barrier()

    base = s * _EPW
    zv = jnp.zeros((_L,), jnp.float32)
    init = (zv,) * 8

    def fetch_idx(bidx, st):
        srco, dstr = st[0], st[1]
        isem = st[7]
        off = base + bidx * _B
        pltpu.async_copy(src.at[pl.ds(off, _B)], srco.at[0], isem)
        pltpu.async_copy(dst.at[pl.ds(off, _B)], dstr.at[0], isem)

    def wait_idx(st):
        srco, dstr = st[0], st[1]
        isem = st[7]
        pltpu.make_async_copy(src.at[pl.ds(0, _B)], srco.at[0], isem).wait()
        pltpu.make_async_copy(dst.at[pl.ds(0, _B)], dstr.at[0], isem).wait()

    def gathers(bidx, st):
        srco, dstr, db_v, eh_v, en_v, g = st[:6]
        off = base + bidx * _B
        pltpu.async_copy(db2.at[c].at[srco.at[0]], db_v, g)
        pltpu.async_copy(eh2.at[c].at[dstr.at[0]], eh_v, g)
        pltpu.async_copy(ce2.at[c, pl.ds(off, _B)], en_v, g)

    def wait_gathers(st):
        srco, dstr, db_v, eh_v, en_v, g = st[:6]
        pltpu.make_async_copy(db2.at[c].at[srco.at[0]], db_v, g).wait()
        pltpu.make_async_copy(eh2.at[c].at[dstr.at[0]], eh_v, g).wait()
        pltpu.make_async_copy(ce2.at[c, pl.ds(0, _B)], en_v, g).wait()

    def put(bidx, st):
        dstr, db_v, en_v, w = st[1], st[2], st[4], st[6]
        off = base + bidx * _B
        pltpu.async_copy(en_v, enew.at[c, pl.ds(off, _B)], w)
        pltpu.sync_copy(db_v, acc.at[dstr.at[0]], add=True)

    def wait_put(st):
        en_v, w = st[4], st[6]
        pltpu.make_async_copy(en_v, enew.at[c, pl.ds(0, _B)], w).wait()

    def compute(st, carry):
        db_v, eh_v, en_v = st[2], st[3], st[4]

        def row(r2, t):
            t = list(t)
            for u in range(2):
                r = r2 * 2 + u
                for j in range(_H // _L):
                    sl = pl.ds(j * _L, _L)
                    sh = pl.ds(_H + j * _L, _L)
                    en = db_v[r, sl] + eh_v[r, sl] + en_v[r, sl]
                    en_v[r, sl] = en
                    sg = 1.0 / (1.0 + jnp.exp(-en))
                    msg = db_v[r, sh] * sg
                    db_v[r, sl] = msg
                    db_v[r, sh] = sg
                    t[j] = t[j] + en
                    t[4 + j] = t[4 + j] + en * en
            return tuple(t)

        return lax.fori_loop(0, _B // 2, row, carry)

    fetch_idx(0, sets[0])
    wait_idx(sets[0])
    gathers(0, sets[0])
    fetch_idx(1, sets[1])

    def outer(i, carry):
        a = 2 * i

        @pl.when(i > 0)
        def _drain1():
            wait_put(sets[1])

        wait_idx(sets[1])
        gathers(a + 1, sets[1])
        wait_gathers(sets[0])

        @pl.when(a + 2 < _NB)
        def _fetch0():
            fetch_idx(a + 2, sets[0])

        carry = compute(sets[0], carry)
        put(a, sets[0])
        wait_put(sets[0])

        @pl.when(a + 2 < _NB)
        def _gather0():
            wait_idx(sets[0])
            gathers(a + 2, sets[0])

        wait_gathers(sets[1])

        @pl.when(a + 3 < _NB)
        def _fetch1():
            fetch_idx(a + 3, sets[1])

        carry = compute(sets[1], carry)
        put(a + 1, sets[1])
        return carry

    carry = lax.fori_loop(0, _NB // 2, outer, init)
    wait_put(sets[1])

    for j in range(_H // _L):
        sums_v[0, pl.ds(j * _L, _L)] = carry[j]
        sums_v[1, pl.ds(j * _L, _L)] = carry[4 + j]
    pltpu.sync_copy(sums_v, stats.at[c, s])

    plsc.subcore_barrier()
    pltpu.sync_copy(acc.at[pl.ds(z0, _NZ)], acc_o.at[c, pl.ds(z0, _NZ)])

    @pl.when(s == _NS - 1)
    def _out_tail():
        t0 = _NS * _NZ
        pltpu.sync_copy(acc.at[pl.ds(t0, _NR)], acc_o.at[c, pl.ds(t0, _NR)])


def _sc_edge(src, dst, db2, eh2, ce2, zeros):
    mesh = plsc.VectorSubcoreMesh(core_axis_name="c", subcore_axis_name="s")
    f = pl.kernel(
        _sc_edge_body,
        out_type=[
            jax.ShapeDtypeStruct((_NC, _E, _H), jnp.float32),
            jax.ShapeDtypeStruct((_NC, _N, _D), jnp.float32),
            jax.ShapeDtypeStruct((_NC, _NS, 2, _H), jnp.float32),
        ],
        mesh=mesh,
        scratch_types=(
            [pltpu.VMEM((1, _B), jnp.int32),
             pltpu.VMEM((1, _B), jnp.int32),
             pltpu.VMEM((_B, _D), jnp.float32),
             pltpu.VMEM((_B, _D), jnp.float32),
             pltpu.VMEM((_B, _H), jnp.float32)] * 2
            + [pltpu.VMEM((2, _H), jnp.float32),
               pltpu.VMEM_SHARED((_N, _D), jnp.float32)]
            + [pltpu.SemaphoreType.DMA] * 6
        ),
    )
    return f(src, dst, db2, eh2, ce2, zeros)


def _hfin_body(ah_ref, acc_ref, h_ref, gh_ref, bh_ref, st_ref,
               ge_ref, be_ref, ho_ref, scale_ref, shift_ref):
    acch = jnp.concatenate([acc_ref[0, :, :_H], acc_ref[1, :, :_H]], axis=1)
    accs = jnp.concatenate([acc_ref[0, :, _H:], acc_ref[1, :, _H:]], axis=1)
    q = ah_ref[...] + acch / (accs + 1e-6)
    m = jnp.mean(q, axis=0, keepdims=True)
    v = jnp.mean((q - m) * (q - m), axis=0, keepdims=True)
    hn = gh_ref[...] * (q - m) / jnp.sqrt(v + 1e-5) + bh_ref[...]
    ho_ref[...] = h_ref[...] + jnp.maximum(hn, 0.0)

    stc = jnp.sum(st_ref[...], axis=1)
    mean = jnp.concatenate([stc[0, 0], stc[1, 0]]) * (1.0 / _E)
    msq = jnp.concatenate([stc[0, 1], stc[1, 1]]) * (1.0 / _E)
    var = msq - mean * mean
    scale = ge_ref[...] / jnp.sqrt(var + 1e-5)
    scale_ref[...] = scale.reshape(1, _D)
    shift_ref[...] = (be_ref[...] - mean * scale).reshape(1, _D)


def _hfin(ah, acc, h, gamma_h, beta_h, stats, gamma_e, beta_e):
    out_shape = [
        jax.ShapeDtypeStruct((_N, _D), jnp.float32),
        jax.ShapeDtypeStruct((1, _D), jnp.float32),
        jax.ShapeDtypeStruct((1, _D), jnp.float32),
    ]
    return pl.pallas_call(_hfin_body, out_shape=out_shape)(
        ah, acc, h, gamma_h, beta_h, stats, gamma_e, beta_e)


_BF = 2000


def _efin_body(e_ref, en_ref, scale_ref, shift_ref, eo_ref):
    en = jnp.concatenate([en_ref[0], en_ref[1]], axis=1)
    eo_ref[...] = e_ref[...] + jnp.maximum(
        en * scale_ref[...] + shift_ref[...], 0.0)


def _efin(e, enew, scale, shift):
    return pl.pallas_call(
        _efin_body,
        grid=(_E // _BF,),
        in_specs=[
            pl.BlockSpec((_BF, _D), lambda i: (i, 0)),
            pl.BlockSpec((_NC, _BF, _H), lambda i: (0, i, 0)),
            pl.BlockSpec((1, _D), lambda i: (0, 0)),
            pl.BlockSpec((1, _D), lambda i: (0, 0)),
        ],
        out_specs=pl.BlockSpec((_BF, _D), lambda i: (i, 0)),
        out_shape=jax.ShapeDtypeStruct((_E, _D), jnp.float32),
    )(e, enew, scale, shift)


def kernel(h, e, edge_index, WA, bA, WB, bB, WC, bC, WD, bD, WE, bE,
           gamma_h, beta_h, gamma_e, beta_e):
    src = edge_index[0]
    dst = edge_index[1]
    zeros = jnp.zeros((_ZB, _D), jnp.float32)

    ah, db2, eh2 = _node_mm(h, WA, bA, WB, bB, WD, bD, WE, bE)
    ce2 = _ce_mm(e, WC, bC)

    enew, acc, stats = _sc_edge(src, dst, db2, eh2, ce2, zeros)

    h_out, scale, shift = _hfin(ah, acc, h, gamma_h, beta_h,
                                stats, gamma_e, beta_e)
    e_out = _efin(e, enew, scale, shift)
    return (h_out, e_out)
